# Initial kernel scaffold; baseline (speedup 1.0000x reference)
import jax
import jax.numpy as jnp
from jax.experimental import pallas as pl


def kernel(x, edge_index, batch, x_mord, Wl1, bl1, Wr1, Wl2, bl2, Wr2, Wl3, bl3, Wr3, fc1_w, fc1_b, bn1_g, bn1_b, fc2_w, fc2_b, bn2_g, bn2_b, fc3_w, fc3_b, bn3_g, bn3_b, lin_w, lin_b):
    raise NotImplementedError("write your pallas kernel here")


# speedup vs baseline: 9.1251x; 9.1251x over previous
import functools

import jax
import jax.numpy as jnp
from jax import lax
from jax.experimental import pallas as pl
from jax.experimental.pallas import tpu as pltpu
from jax.experimental.pallas import tpu_sc as plsc

N_NODES = 100000
N_EDGES = 1600000
N_GRAPHS = 1024
MAX_DEG = 10
EPS = 1e-5

NC, NS = 2, 16
SUP = 8
CHUNK = 128
EPAD = 1605632
NROW = 102400
NTILE = NROW // NS

_mesh = lambda: plsc.VectorSubcoreMesh(
    core_axis_name="c", subcore_axis_name="s", num_cores=NC, num_subcores=NS)


def _zero_rows(buf, nrows):
    def z(i, c):
        buf[i] = jnp.zeros((16,), jnp.float32)
        return c
    lax.fori_loop(0, nrows, z, 0)


def _make_seg(split_edges):
    n_tiles = NC * NS if split_edges else NS
    chunks_per_tile = EPAD // (n_tiles * CHUNK)
    nsup = chunks_per_tile // SUP

    def body(*refs):
        if split_edges:
            ta, src2, dst2, out, src_m, dst_m, rows, zbuf, acc, gsem = refs
            tb = ta
        else:
            ta, tb, src2, dst2, out, src_m, dst_m, rows, zbuf, acc, gsem = refs
        cid = lax.axis_index("c")
        sid = lax.axis_index("s")

        _zero_rows(zbuf, 128)
        def za(k, c):
            pltpu.sync_copy(zbuf, acc.at[pl.ds(sid * NTILE + k * 128, 128)])
            return c
        lax.fori_loop(0, NTILE // 128, za, 0)
        plsc.subcore_barrier()

        def run(table, base_chunk):
            def sup(s, c):
                row0 = base_chunk + s * SUP
                pltpu.sync_copy(src2.at[pl.ds(row0, SUP)], src_m)
                pltpu.sync_copy(dst2.at[pl.ds(row0, SUP)], dst_m)
                descs = [pltpu.async_copy(table.at[src_m.at[j]], rows.at[j], gsem)
                         for j in range(SUP)]
                for d in descs:
                    d.wait()
                for j in range(SUP):
                    pltpu.sync_copy(rows.at[j], acc.at[dst_m.at[j]], add=True)
                return c
            lax.fori_loop(0, nsup, sup, 0)

        if split_edges:
            run(ta, (cid * NS + sid) * chunks_per_tile)
        else:
            @pl.when(cid == 0)
            def _():
                run(ta, sid * chunks_per_tile)
            @pl.when(cid == 1)
            def _():
                run(tb, sid * chunks_per_tile)

        plsc.subcore_barrier()
        pltpu.sync_copy(acc.at[pl.ds(sid * NTILE, NTILE)],
                        out.at[cid].at[pl.ds(sid * NTILE, NTILE)])

    return pl.kernel(
        body,
        out_type=jax.ShapeDtypeStruct((2, NROW, 16), jnp.float32),
        mesh=_mesh(),
        compiler_params=pltpu.CompilerParams(use_tc_tiling_on_sc=False),
        scratch_types=[
            pltpu.VMEM((SUP, CHUNK), jnp.int32),
            pltpu.VMEM((SUP, CHUNK), jnp.int32),
            pltpu.VMEM((SUP, CHUNK, 16), jnp.float32),
            pltpu.VMEM((128, 16), jnp.float32),
            pltpu.VMEM_SHARED((NROW, 16), jnp.float32),
            pltpu.SemaphoreType.DMA,
        ],
    )


NGACC = N_GRAPHS + 8


def _make_pool():
    PC = 128
    CPW = 25

    def body(ha, hb, b2, oa, ob, oc, bidx, bufa, bufb, onesb, zbuf,
             acca, accb, accc):
        cid = lax.axis_index("c")
        sid = lax.axis_index("s")
        w = sid * NC + cid

        i16 = lax.iota(jnp.int32, 16)
        e0 = jnp.where(i16 == 0, jnp.float32(1.0), jnp.float32(0.0))
        def onit(i, c):
            onesb[i] = e0
            return c
        lax.fori_loop(0, PC, onit, 0)
        _zero_rows(zbuf, 72)
        pltpu.sync_copy(zbuf.at[pl.ds(0, 64)], acca.at[pl.ds(sid * 64, 64)])
        pltpu.sync_copy(zbuf.at[pl.ds(0, 64)], accb.at[pl.ds(sid * 64, 64)])
        pltpu.sync_copy(zbuf.at[pl.ds(0, 64)], accc.at[pl.ds(sid * 64, 64)])
        @pl.when(sid == 0)
        def _():
            pltpu.sync_copy(zbuf.at[pl.ds(64, 8)], acca.at[pl.ds(N_GRAPHS, 8)])
            pltpu.sync_copy(zbuf.at[pl.ds(64, 8)], accb.at[pl.ds(N_GRAPHS, 8)])
            pltpu.sync_copy(zbuf.at[pl.ds(64, 8)], accc.at[pl.ds(N_GRAPHS, 8)])
        pltpu.sync_copy(b2, bidx)
        plsc.subcore_barrier()

        def ck(k, c):
            row = w * CPW + k
            base = row * PC
            pltpu.sync_copy(ha.at[pl.ds(base, PC)], bufa)
            pltpu.sync_copy(hb.at[pl.ds(base, PC)], bufb)
            pltpu.sync_copy(bufa, acca.at[bidx.at[row]], add=True)
            pltpu.sync_copy(bufb, accb.at[bidx.at[row]], add=True)
            pltpu.sync_copy(onesb, accc.at[bidx.at[row]], add=True)
            return c
        lax.fori_loop(0, CPW, ck, 0)
        plsc.subcore_barrier()

        pltpu.sync_copy(acca.at[pl.ds(sid * 64, 64)],
                        oa.at[cid].at[pl.ds(sid * 64, 64)])
        pltpu.sync_copy(accb.at[pl.ds(sid * 64, 64)],
                        ob.at[cid].at[pl.ds(sid * 64, 64)])
        pltpu.sync_copy(accc.at[pl.ds(sid * 64, 64)],
                        oc.at[cid].at[pl.ds(sid * 64, 64)])

    st = jax.ShapeDtypeStruct((2, N_GRAPHS, 16), jnp.float32)
    return pl.kernel(
        body,
        out_type=(st, st, st),
        mesh=_mesh(),
        compiler_params=pltpu.CompilerParams(use_tc_tiling_on_sc=False),
        scratch_types=[
            pltpu.VMEM((NROW // 128, 128), jnp.int32),
            pltpu.VMEM((PC, 16), jnp.float32),
            pltpu.VMEM((PC, 16), jnp.float32),
            pltpu.VMEM((PC, 16), jnp.float32),
            pltpu.VMEM((72, 16), jnp.float32),
            pltpu.VMEM_SHARED((NGACC, 16), jnp.float32),
            pltpu.VMEM_SHARED((NGACC, 16), jnp.float32),
            pltpu.VMEM_SHARED((NGACC, 16), jnp.float32),
        ],
    )


_BD = 1600


def _dense1_body(pref, xref, wref, bref, haref, hbref, degref):
    ps = pref[0] + pref[1]
    degf = jnp.clip(ps[:, 8:9], 0.0, 10.0)
    cat = jnp.concatenate([ps[:, 0:8], xref[...]], axis=1)
    z = jnp.dot(cat, wref[...], preferred_element_type=jnp.float32)
    acc = jnp.zeros((_BD, 32), jnp.float32)
    for d in range(MAX_DEG + 1):
        m = (degf == float(d)).astype(jnp.float32)
        acc = acc + m * (z[:, d * 32:(d + 1) * 32] + bref[d])
    out = jnp.maximum(acc, 0.0)
    haref[...] = out[:, :16]
    hbref[...] = out[:, 16:]
    degref[...] = degf


def _dense1(p, x, wcat, bl):
    grid = (NROW // _BD,)
    return pl.pallas_call(
        _dense1_body,
        grid=grid,
        in_specs=[
            pl.BlockSpec((2, _BD, 16), lambda i: (0, i, 0)),
            pl.BlockSpec((_BD, 8), lambda i: (i, 0)),
            pl.BlockSpec((16, 352), lambda i: (0, 0)),
            pl.BlockSpec((MAX_DEG + 1, 32), lambda i: (0, 0)),
        ],
        out_specs=[
            pl.BlockSpec((_BD, 16), lambda i: (i, 0)),
            pl.BlockSpec((_BD, 16), lambda i: (i, 0)),
            pl.BlockSpec((_BD, 1), lambda i: (i, 0)),
        ],
        out_shape=[
            jax.ShapeDtypeStruct((NROW, 16), jnp.float32),
            jax.ShapeDtypeStruct((NROW, 16), jnp.float32),
            jax.ShapeDtypeStruct((NROW, 1), jnp.float32),
        ],
    )(p, x, wcat, bl)


def _dense23_body(relu, pref, xaref, xbref, degref, wref, bref, haref, hbref):
    cat = jnp.concatenate(
        [pref[0], pref[1], xaref[...], xbref[...]], axis=1)
    z = jnp.dot(cat, wref[...], preferred_element_type=jnp.float32)
    degf = degref[...]
    acc = jnp.zeros((_BD, 32), jnp.float32)
    for d in range(MAX_DEG + 1):
        m = (degf == float(d)).astype(jnp.float32)
        acc = acc + m * (z[:, d * 32:(d + 1) * 32] + bref[d])
    out = jnp.maximum(acc, 0.0) if relu else acc
    haref[...] = out[:, :16]
    hbref[...] = out[:, 16:]


def _dense23(p, xa, xb, deg, wcat, bl, relu):
    grid = (NROW // _BD,)
    return pl.pallas_call(
        functools.partial(_dense23_body, relu),
        grid=grid,
        in_specs=[
            pl.BlockSpec((2, _BD, 16), lambda i: (0, i, 0)),
            pl.BlockSpec((_BD, 16), lambda i: (i, 0)),
            pl.BlockSpec((_BD, 16), lambda i: (i, 0)),
            pl.BlockSpec((_BD, 1), lambda i: (i, 0)),
            pl.BlockSpec((64, 352), lambda i: (0, 0)),
            pl.BlockSpec((MAX_DEG + 1, 32), lambda i: (0, 0)),
        ],
        out_specs=[
            pl.BlockSpec((_BD, 16), lambda i: (i, 0)),
            pl.BlockSpec((_BD, 16), lambda i: (i, 0)),
        ],
        out_shape=[
            jax.ShapeDtypeStruct((NROW, 16), jnp.float32),
            jax.ShapeDtypeStruct((NROW, 16), jnp.float32),
        ],
    )(p, xa, xb, deg, wcat, bl)


def _head_body(oaref, obref, ocref, xmref, w1ref, b1ref, s1ref, t1ref,
               w2ref, b2ref, s2ref, t2ref, w3ref, b3ref, s3ref, t3ref,
               lwref, lbref, outref):
    A = oaref[0] + oaref[1]
    B = obref[0] + obref[1]
    C = ocref[0] + ocref[1]
    cnt = jnp.maximum(C[:, 0:1], 1.0)
    g = jnp.concatenate([A, B], axis=1) / cnt
    m = jnp.dot(xmref[...], w1ref[...], preferred_element_type=jnp.float32)
    m = jnp.maximum(m + b1ref[...], 0.0) * s1ref[...] + t1ref[...]
    m = jnp.dot(m, w2ref[...], preferred_element_type=jnp.float32)
    m = jnp.maximum(m + b2ref[...], 0.0) * s2ref[...] + t2ref[...]
    m = jnp.dot(m, w3ref[...], preferred_element_type=jnp.float32)
    m = jnp.maximum(m + b3ref[...], 0.0) * s3ref[...] + t3ref[...]
    lw = lwref[...]
    z = (jnp.dot(g, lw[:32], preferred_element_type=jnp.float32)
         + jnp.dot(m, lw[32:], preferred_element_type=jnp.float32)
         + lbref[...])
    outref[...] = 1.0 / (1.0 + jnp.exp(-z))


def _head(oa, ob, oc, xm, w1, b1, s1, t1, w2, b2, s2, t2, w3, b3, s3, t3,
          lw, lb):
    return pl.pallas_call(
        _head_body,
        out_shape=jax.ShapeDtypeStruct((N_GRAPHS, 1), jnp.float32),
    )(oa, ob, oc, xm, w1, b1, s1, t1, w2, b2, s2, t2, w3, b3, s3, t3, lw, lb)


def _wcat(wl, wr):
    w = jnp.concatenate([wl, wr], axis=1)
    return jnp.transpose(w, (1, 0, 2)).reshape(w.shape[1], 11 * 32)


def kernel(x, edge_index, batch, x_mord, Wl1, bl1, Wr1, Wl2, bl2, Wr2,
           Wl3, bl3, Wr3, fc1_w, fc1_b, bn1_g, bn1_b, fc2_w, fc2_b, bn2_g,
           bn2_b, fc3_w, fc3_b, bn3_g, bn3_b, lin_w, lin_b):
    pad = EPAD - N_EDGES
    src2 = jnp.concatenate(
        [edge_index[0], jnp.zeros((pad,), jnp.int32)]).reshape(EPAD // CHUNK, CHUNK)
    dst2 = jnp.concatenate(
        [edge_index[1], jnp.full((pad,), N_NODES, jnp.int32)]).reshape(EPAD // CHUNK, CHUNK)
    xpad = jnp.concatenate(
        [x, jnp.ones((N_NODES, 1), jnp.float32),
         jnp.zeros((N_NODES, 7), jnp.float32)], axis=1)
    xdense = jnp.concatenate(
        [x, jnp.zeros((NROW - N_NODES, 8), jnp.float32)], axis=0)

    seg1 = _make_seg(split_edges=True)
    seg23 = _make_seg(split_edges=False)

    p1 = seg1(xpad, src2, dst2)
    ha1, hb1, deg = _dense1(p1, xdense, _wcat(Wl1, Wr1), bl1)
    p2 = seg23(ha1, hb1, src2, dst2)
    ha2, hb2 = _dense23(p2, ha1, hb1, deg, _wcat(Wl2, Wr2), bl2, relu=True)
    p3 = seg23(ha2, hb2, src2, dst2)
    ha3, hb3 = _dense23(p3, ha2, hb2, deg, _wcat(Wl3, Wr3), bl3, relu=False)

    bpad = jnp.concatenate(
        [batch, jnp.full((NROW - N_NODES,), N_GRAPHS, jnp.int32)])
    oa, ob, oc = _make_pool()(ha3, hb3, bpad.reshape(NROW // 128, 128))

    inv = 1.0 / jnp.sqrt(1.0 + EPS)
    return _head(oa, ob, oc, x_mord,
                 fc1_w, fc1_b, bn1_g * inv, bn1_b,
                 fc2_w, fc2_b, bn2_g * inv, bn2_b,
                 fc3_w, fc3_b, bn3_g * inv, bn3_b,
                 lin_w, lin_b)
